# Initial kernel scaffold; baseline (speedup 1.0000x reference)
import jax
import jax.numpy as jnp
from jax.experimental import pallas as pl


def kernel(x, edge_index, W1, b1, W2, b2):
    raise NotImplementedError("write your pallas kernel here")


# speedup vs baseline: 68.5984x; 68.5984x over previous
import functools

import jax
import jax.numpy as jnp
from jax import lax
from jax.experimental import pallas as pl
from jax.experimental.pallas import tpu as pltpu
from jax.experimental.pallas import tpu_sc as plsc

N = 10000
E = 320000
DF = 128
DH = 16

NPAD = 10240
NSLICE = NPAD // 16
BATCH = 80
ROWS = E // BATCH
ROWS_PER_CORE = ROWS // 2
ROWS_PER_TILE = ROWS_PER_CORE // 16

_mesh = functools.partial(
    plsc.VectorSubcoreMesh, core_axis_name="c", subcore_axis_name="s"
)


@functools.partial(
    pl.kernel,
    out_type=jax.ShapeDtypeStruct((2, NPAD), jnp.float32),
    mesh=_mesh(),
    scratch_types=[
        pltpu.VMEM((ROWS_PER_TILE, BATCH), jnp.int32),
        pltpu.VMEM((BATCH,), jnp.float32),
        pltpu.VMEM((NSLICE,), jnp.float32),
        pltpu.VMEM_SHARED((NPAD,), jnp.float32),
        pltpu.SemaphoreType.DMA,
    ],
)
def _deg_kernel(dst2_hbm, deg_hbm, dst_v, ones_v, buf_v, deg_sh, sem):
    c = lax.axis_index("c")
    s = lax.axis_index("s")
    blk = c * 16 + s
    pltpu.async_copy(dst2_hbm.at[blk], dst_v, sem).wait()
    for i in range(BATCH // 16):
        ones_v[pl.ds(i * 16, 16)] = jnp.ones((16,), jnp.float32)
    for i in range(NSLICE // 16):
        buf_v[pl.ds(i * 16, 16)] = jnp.zeros((16,), jnp.float32)
    pltpu.sync_copy(buf_v, deg_sh.at[pl.ds(s * NSLICE, NSLICE)])
    plsc.subcore_barrier()

    def body(j, carry):
        pltpu.sync_copy(ones_v, deg_sh.at[dst_v.at[j]], add=True)
        return carry

    lax.fori_loop(0, ROWS_PER_TILE, body, 0)
    plsc.subcore_barrier()
    pltpu.sync_copy(deg_sh.at[pl.ds(s * NSLICE, NSLICE)], buf_v)
    pltpu.sync_copy(buf_v, deg_hbm.at[c, pl.ds(s * NSLICE, NSLICE)])


def _prep_body(xp_ref, w1_ref, degp_ref, y_ref, dinv_ref):
    deg = degp_ref[0, :] + degp_ref[1, :] + 1.0
    dinv = lax.rsqrt(deg)
    xw = jnp.dot(xp_ref[...], w1_ref[...], preferred_element_type=jnp.float32)
    y_ref[...] = xw * dinv[:, None]
    dinv_ref[...] = dinv


_prep = pl.pallas_call(
    _prep_body,
    out_shape=(
        jax.ShapeDtypeStruct((NPAD, DH), jnp.float32),
        jax.ShapeDtypeStruct((NPAD,), jnp.float32),
    ),
)


@functools.partial(
    pl.kernel,
    out_type=(
        jax.ShapeDtypeStruct((2, NPAD, DH), jnp.float32),
        jax.ShapeDtypeStruct((2, NPAD), jnp.float32),
    ),
    mesh=_mesh(),
    scratch_types=[
        pltpu.VMEM((ROWS_PER_TILE, BATCH), jnp.int32),
        pltpu.VMEM((ROWS_PER_TILE, BATCH), jnp.int32),
        pltpu.VMEM((BATCH, DH), jnp.float32),
        pltpu.VMEM((BATCH,), jnp.float32),
        pltpu.VMEM((NSLICE, DH), jnp.float32),
        pltpu.VMEM((NSLICE,), jnp.float32),
        pltpu.VMEM_SHARED((NPAD, DH), jnp.float32),
        pltpu.VMEM_SHARED((NPAD,), jnp.float32),
        pltpu.VMEM_SHARED((NPAD, DH), jnp.float32),
        pltpu.VMEM_SHARED((NPAD,), jnp.float32),
        pltpu.SemaphoreType.DMA,
        pltpu.SemaphoreType.DMA,
    ],
    compiler_params=pltpu.CompilerParams(use_tc_tiling_on_sc=False),
)
def _agg_kernel(
    y_hbm, dinv_hbm, src2_hbm, dst2_hbm,
    agg_hbm, q_hbm,
    src_v, dst_v, rows_v, dval_v, buf16_v, buf_v,
    agg_sh, q_sh, y_sh, dinv_sh, sem, sem2,
):
    c = lax.axis_index("c")
    s = lax.axis_index("s")
    blk = c * 16 + s
    pltpu.async_copy(src2_hbm.at[blk], src_v, sem).wait()
    pltpu.async_copy(dst2_hbm.at[blk], dst_v, sem).wait()

    nsl = pl.ds(s * NSLICE, NSLICE)
    pltpu.async_copy(y_hbm.at[nsl], buf16_v, sem).wait()
    pltpu.sync_copy(buf16_v, y_sh.at[nsl])
    pltpu.async_copy(dinv_hbm.at[nsl], buf_v, sem).wait()
    pltpu.sync_copy(buf_v, dinv_sh.at[nsl])

    def zero16(i, carry):
        buf16_v[i, :] = jnp.zeros((16,), jnp.float32)
        return carry

    lax.fori_loop(0, NSLICE, zero16, 0)
    for i in range(NSLICE // 16):
        buf_v[pl.ds(i * 16, 16)] = jnp.zeros((16,), jnp.float32)
    pltpu.sync_copy(buf16_v, agg_sh.at[nsl])
    pltpu.sync_copy(buf_v, q_sh.at[nsl])
    plsc.subcore_barrier()

    def body(j, carry):
        pltpu.async_copy(y_sh.at[src_v.at[j]], rows_v, sem).wait()
        pltpu.async_copy(dinv_sh.at[dst_v.at[j]], dval_v, sem2).wait()
        pltpu.sync_copy(rows_v, agg_sh.at[dst_v.at[j]], add=True)
        pltpu.sync_copy(dval_v, q_sh.at[src_v.at[j]], add=True)
        return carry

    lax.fori_loop(0, ROWS_PER_TILE, body, 0)
    plsc.subcore_barrier()
    pltpu.sync_copy(agg_sh.at[pl.ds(s * NSLICE, NSLICE)], buf16_v)
    pltpu.sync_copy(buf16_v, agg_hbm.at[c, pl.ds(s * NSLICE, NSLICE)])
    pltpu.sync_copy(q_sh.at[pl.ds(s * NSLICE, NSLICE)], buf_v)
    pltpu.sync_copy(buf_v, q_hbm.at[c, pl.ds(s * NSLICE, NSLICE)])


def _final_body(aggp_ref, y_ref, dinv_ref, qp_ref, b1_ref, w2_ref, b2_ref,
                out_ref):
    dinv = dinv_ref[...]
    agg = aggp_ref[0] + aggp_ref[1] + y_ref[...]
    h = agg * dinv[:, None] + b1_ref[...][None, :]
    g = jnp.maximum(h, 0.0)
    z = jnp.dot(g, w2_ref[...], preferred_element_type=jnp.float32)[:, 0]
    q = qp_ref[0] + qp_ref[1] + dinv
    t = dinv * q
    mask = (lax.broadcasted_iota(jnp.int32, (NPAD,), 0) < N).astype(
        jnp.float32
    )
    val = jnp.sum(z * t * mask) * (1.0 / N) + b2_ref[0]
    out_ref[...] = jnp.reshape(val, (1, 1))


_final = pl.pallas_call(
    _final_body,
    out_shape=jax.ShapeDtypeStruct((1, 1), jnp.float32),
)


def kernel(x, edge_index, W1, b1, W2, b2):
    src2 = edge_index[0].reshape(32, ROWS_PER_TILE, BATCH)
    dst2 = edge_index[1].reshape(32, ROWS_PER_TILE, BATCH)
    xp = jnp.zeros((NPAD, DF), jnp.float32).at[:N].set(x)
    degp = _deg_kernel(dst2)
    y, dinv = _prep(xp, W1, degp)
    aggp, qp = _agg_kernel(y, dinv, src2, dst2)
    out = _final(aggp, y, dinv, qp, b1, W2, b2)
    return out[0, 0]


# speedup vs baseline: 98.4937x; 1.4358x over previous
import functools

import jax
import jax.numpy as jnp
from jax import lax
from jax.experimental import pallas as pl
from jax.experimental.pallas import tpu as pltpu
from jax.experimental.pallas import tpu_sc as plsc

N = 10000
E = 320000
DF = 128
DH = 16

NPAD = 10240
NSLICE = NPAD // 16
BATCH = 125
ITERS = 80
NBUF = 4
LOOKAHEAD = 2

_mesh = functools.partial(
    plsc.VectorSubcoreMesh, core_axis_name="c", subcore_axis_name="s"
)


@functools.partial(
    pl.kernel,
    out_type=jax.ShapeDtypeStruct((2, NPAD), jnp.float32),
    mesh=_mesh(),
    scratch_types=[
        pltpu.VMEM((ITERS, BATCH), jnp.int32),
        pltpu.VMEM((128,), jnp.float32),
        pltpu.VMEM((NSLICE,), jnp.float32),
        pltpu.VMEM_SHARED((NPAD,), jnp.float32),
        pltpu.SemaphoreType.DMA,
        pltpu.SemaphoreType.DMA,
    ],
    compiler_params=pltpu.CompilerParams(use_tc_tiling_on_sc=False),
)
def _deg_kernel(dst2_hbm, deg_hbm, dst_v, ones_v, buf_v, deg_sh, sem, ssem):
    c = lax.axis_index("c")
    s = lax.axis_index("s")
    blk = c * 16 + s
    pltpu.async_copy(dst2_hbm.at[blk], dst_v, sem).wait()
    for i in range(8):
        ones_v[pl.ds(i * 16, 16)] = jnp.ones((16,), jnp.float32)
    for i in range(NSLICE // 16):
        buf_v[pl.ds(i * 16, 16)] = jnp.zeros((16,), jnp.float32)
    pltpu.sync_copy(buf_v, deg_sh.at[pl.ds(s * NSLICE, NSLICE)])
    plsc.subcore_barrier()

    K = 20
    for base in range(0, ITERS, K):
        descs = [
            pltpu.async_copy(
                ones_v.at[pl.ds(0, BATCH)],
                deg_sh.at[dst_v.at[base + j]],
                ssem,
                add=True,
            )
            for j in range(K)
        ]
        for d in descs:
            d.wait()
    plsc.subcore_barrier()
    pltpu.sync_copy(deg_sh.at[pl.ds(s * NSLICE, NSLICE)], buf_v)
    pltpu.sync_copy(buf_v, deg_hbm.at[c, pl.ds(s * NSLICE, NSLICE)])


def _prep_body(xp_ref, w1_ref, degp_ref, y_ref, dinv_ref):
    deg = degp_ref[0, :] + degp_ref[1, :] + 1.0
    dinv = lax.rsqrt(deg)
    xw = jnp.dot(xp_ref[...], w1_ref[...], preferred_element_type=jnp.float32)
    y_ref[...] = xw * dinv[:, None]
    dinv_ref[...] = dinv


_prep = pl.pallas_call(
    _prep_body,
    out_shape=(
        jax.ShapeDtypeStruct((NPAD, DH), jnp.float32),
        jax.ShapeDtypeStruct((NPAD,), jnp.float32),
    ),
)


@functools.partial(
    pl.kernel,
    out_type=(
        jax.ShapeDtypeStruct((2, NPAD, DH), jnp.float32),
        jax.ShapeDtypeStruct((2, NPAD), jnp.float32),
    ),
    mesh=_mesh(),
    scratch_types=[
        pltpu.VMEM((ITERS, BATCH), jnp.int32),
        pltpu.VMEM((ITERS, BATCH), jnp.int32),
        pltpu.VMEM((NBUF, BATCH, DH), jnp.float32),
        pltpu.VMEM((NBUF, BATCH), jnp.float32),
        pltpu.VMEM((NSLICE, DH), jnp.float32),
        pltpu.VMEM((NSLICE,), jnp.float32),
        pltpu.VMEM_SHARED((NPAD, DH), jnp.float32),
        pltpu.VMEM_SHARED((NPAD,), jnp.float32),
        pltpu.VMEM_SHARED((NPAD, DH), jnp.float32),
        pltpu.VMEM_SHARED((NPAD,), jnp.float32),
        pltpu.SemaphoreType.DMA,
        [pltpu.SemaphoreType.DMA] * NBUF,
        [pltpu.SemaphoreType.DMA] * NBUF,
    ],
    compiler_params=pltpu.CompilerParams(use_tc_tiling_on_sc=False),
)
def _agg_kernel(
    y_hbm, dinv_hbm, src2_hbm, dst2_hbm,
    agg_hbm, q_hbm,
    src_v, dst_v, rows_v, dval_v, buf16_v, buf_v,
    agg_sh, q_sh, y_sh, dinv_sh, sem, gsems, ssems,
):
    c = lax.axis_index("c")
    s = lax.axis_index("s")
    blk = c * 16 + s
    pltpu.async_copy(src2_hbm.at[blk], src_v, sem).wait()
    pltpu.async_copy(dst2_hbm.at[blk], dst_v, sem).wait()

    nsl = pl.ds(s * NSLICE, NSLICE)
    pltpu.async_copy(y_hbm.at[nsl], buf16_v, sem).wait()
    pltpu.sync_copy(buf16_v, y_sh.at[nsl])
    pltpu.async_copy(dinv_hbm.at[nsl], buf_v, sem).wait()
    pltpu.sync_copy(buf_v, dinv_sh.at[nsl])

    def zero16(i, carry):
        buf16_v[i, :] = jnp.zeros((16,), jnp.float32)
        return carry

    lax.fori_loop(0, NSLICE, zero16, 0)
    for i in range(NSLICE // 16):
        buf_v[pl.ds(i * 16, 16)] = jnp.zeros((16,), jnp.float32)
    pltpu.sync_copy(buf16_v, agg_sh.at[nsl])
    pltpu.sync_copy(buf_v, q_sh.at[nsl])
    plsc.subcore_barrier()

    gd = {}
    sd = {}

    def issue_g(j):
        p = j % NBUF
        gd[j] = (
            pltpu.async_copy(y_sh.at[src_v.at[j]], rows_v.at[p], gsems[p]),
            pltpu.async_copy(dinv_sh.at[dst_v.at[j]], dval_v.at[p], gsems[p]),
        )

    def issue_s(j):
        p = j % NBUF
        sd[j] = (
            pltpu.async_copy(
                rows_v.at[p], agg_sh.at[dst_v.at[j]], ssems[p], add=True
            ),
            pltpu.async_copy(
                dval_v.at[p], q_sh.at[src_v.at[j]], ssems[p], add=True
            ),
        )

    for j in range(LOOKAHEAD):
        issue_g(j)
    for j in range(ITERS):
        nxt = j + LOOKAHEAD
        if nxt < ITERS:
            prev = nxt - NBUF
            if prev >= 0:
                sd[prev][0].wait()
                sd[prev][1].wait()
            issue_g(nxt)
        gd[j][0].wait()
        gd[j][1].wait()
        issue_s(j)
    for j in range(ITERS - NBUF, ITERS):
        sd[j][0].wait()
        sd[j][1].wait()

    plsc.subcore_barrier()
    pltpu.sync_copy(agg_sh.at[nsl], buf16_v)
    pltpu.sync_copy(buf16_v, agg_hbm.at[c, nsl])
    pltpu.sync_copy(q_sh.at[nsl], buf_v)
    pltpu.sync_copy(buf_v, q_hbm.at[c, nsl])


def _final_body(aggp_ref, y_ref, dinv_ref, qp_ref, b1_ref, w2_ref, b2_ref,
                out_ref):
    dinv = dinv_ref[...]
    agg = aggp_ref[0] + aggp_ref[1] + y_ref[...]
    h = agg * dinv[:, None] + b1_ref[...][None, :]
    g = jnp.maximum(h, 0.0)
    z = jnp.dot(g, w2_ref[...], preferred_element_type=jnp.float32)[:, 0]
    q = qp_ref[0] + qp_ref[1] + dinv
    t = dinv * q
    mask = (lax.broadcasted_iota(jnp.int32, (NPAD,), 0) < N).astype(
        jnp.float32
    )
    val = jnp.sum(z * t * mask) * (1.0 / N) + b2_ref[0]
    out_ref[...] = jnp.reshape(val, (1, 1))


_final = pl.pallas_call(
    _final_body,
    out_shape=jax.ShapeDtypeStruct((1, 1), jnp.float32),
)


def kernel(x, edge_index, W1, b1, W2, b2):
    src2 = edge_index[0].reshape(32, ITERS, BATCH)
    dst2 = edge_index[1].reshape(32, ITERS, BATCH)
    xp = jnp.zeros((NPAD, DF), jnp.float32).at[:N].set(x)
    degp = _deg_kernel(dst2)
    y, dinv = _prep(xp, W1, degp)
    aggp, qp = _agg_kernel(y, dinv, src2, dst2)
    out = _final(aggp, y, dinv, qp, b1, W2, b2)
    return out[0, 0]
